# Initial kernel scaffold; baseline (speedup 1.0000x reference)
import jax
import jax.numpy as jnp
from jax.experimental import pallas as pl


def kernel(x, edge_index, W1, b1, W2, b2, W3, b3):
    raise NotImplementedError("write your pallas kernel here")


# speedup vs baseline: 12.0137x; 12.0137x over previous
import functools

import jax
import jax.numpy as jnp
from jax import lax
from jax.experimental import pallas as pl
from jax.experimental.pallas import tpu as pltpu
import jax.experimental.pallas.tpu_sc as plsc

N = 10000
E = 160000
IN_DIM = 256
HID_DIM = 256
LAT_DIM = 128
OUT_DIM = 256

NC = 2
NT = 16
ROWS_PT = N // NT
EPT = E // NT
CH = 125
NCHUNK = EPT // CH
INIT_CHUNKS = ROWS_PT // CH
DEG_EPT = E // (NT * NC)
DEG_CH = 40
DEG_NCHUNK = DEG_EPT // DEG_CH
DEG_SPAN = 632

_MESH = plsc.VectorSubcoreMesh(core_axis_name="c", subcore_axis_name="s")


def _build_deg(interpret=False):
    return functools.partial(
        pl.kernel,
        out_type=jax.ShapeDtypeStruct((NC * N,), jnp.float32),
        mesh=_MESH,
        scratch_types=[
            pltpu.VMEM((DEG_NCHUNK, DEG_CH), jnp.int32),
            pltpu.VMEM((48,), jnp.float32),
            pltpu.VMEM((640,), jnp.float32),
            pltpu.VMEM_SHARED((N,), jnp.float32),
            pltpu.SemaphoreType.DMA,
        ],
        compiler_params=pltpu.CompilerParams(use_tc_tiling_on_sc=False),
        interpret=interpret,
    )(_deg_body)


def _deg_body(dst_hbm, out_hbm, dstbuf, ones_v, zeros_v, acc, sem):
    c = lax.axis_index("c")
    s = lax.axis_index("s")
    w = c * NT + s
    pltpu.sync_copy(dst_hbm.at[w], dstbuf)
    for k in range(48 // 16):
        ones_v[pl.ds(k * 16, 16)] = jnp.ones((16,), jnp.float32)
    for k in range(640 // 16):
        zeros_v[pl.ds(k * 16, 16)] = jnp.zeros((16,), jnp.float32)

    @pl.when(s < NT - 1)
    def _():
        pltpu.sync_copy(zeros_v.at[pl.ds(0, DEG_SPAN)],
                        acc.at[pl.ds(s * DEG_SPAN, DEG_SPAN)])

    @pl.when(s == NT - 1)
    def _():
        pltpu.sync_copy(zeros_v.at[pl.ds(0, N - (NT - 1) * DEG_SPAN)],
                        acc.at[pl.ds((NT - 1) * DEG_SPAN, N - (NT - 1) * DEG_SPAN)])

    plsc.subcore_barrier()

    def chunk(j, carry):
        pltpu.sync_copy(ones_v.at[pl.ds(0, DEG_CH)], acc.at[dstbuf.at[j]], add=True)
        return carry

    lax.fori_loop(0, DEG_NCHUNK, chunk, 0)
    plsc.subcore_barrier()

    @pl.when(s < NT - 1)
    def _():
        pltpu.sync_copy(acc.at[pl.ds(s * DEG_SPAN, DEG_SPAN)],
                        zeros_v.at[pl.ds(0, DEG_SPAN)])
        pltpu.sync_copy(zeros_v.at[pl.ds(0, DEG_SPAN)],
                        out_hbm.at[pl.ds(c * N + s * DEG_SPAN, DEG_SPAN)])

    @pl.when(s == NT - 1)
    def _():
        pltpu.sync_copy(acc.at[pl.ds((NT - 1) * DEG_SPAN, N - (NT - 1) * DEG_SPAN)],
                        zeros_v.at[pl.ds(0, N - (NT - 1) * DEG_SPAN)])
        pltpu.sync_copy(zeros_v.at[pl.ds(0, N - (NT - 1) * DEG_SPAN)],
                        out_hbm.at[pl.ds(c * N + (NT - 1) * DEG_SPAN, N - (NT - 1) * DEG_SPAN)])


def _make_prop(dh, interpret=False):
    @functools.partial(
        pl.kernel,
        out_type=jax.ShapeDtypeStruct((NC * N, dh), jnp.float32),
        mesh=_MESH,
        scratch_types=[
            pltpu.VMEM((NCHUNK, CH), jnp.int32),
            pltpu.VMEM((NCHUNK, CH), jnp.int32),
            pltpu.VMEM((CH, dh), jnp.float32),
            pltpu.VMEM_SHARED((N, dh), jnp.float32),
            pltpu.SemaphoreType.DMA,
        ],
        compiler_params=pltpu.CompilerParams(use_tc_tiling_on_sc=False),
        interpret=interpret,
    )
    def prop(y_hbm, srcb_hbm, dstb_hbm, out_hbm, srcb, dstb, rows, acc, sem):
        c = lax.axis_index("c")
        s = lax.axis_index("s")
        pltpu.sync_copy(srcb_hbm.at[c, s], srcb)
        pltpu.sync_copy(dstb_hbm.at[s], dstb)

        def init(k, carry):
            pltpu.sync_copy(y_hbm.at[pl.ds(c * N + s * ROWS_PT + k * CH, CH)], rows)
            pltpu.sync_copy(rows, acc.at[pl.ds(s * ROWS_PT + k * CH, CH)])
            return carry

        lax.fori_loop(0, INIT_CHUNKS, init, 0)
        plsc.subcore_barrier()

        def chunk(j, carry):
            pltpu.async_copy(y_hbm.at[srcb.at[j]], rows, sem).wait()
            pltpu.sync_copy(rows, acc.at[dstb.at[j]], add=True)
            return carry

        lax.fori_loop(0, NCHUNK, chunk, 0)
        plsc.subcore_barrier()

        def drain(k, carry):
            pltpu.sync_copy(acc.at[pl.ds(s * ROWS_PT + k * CH, CH)], rows)
            pltpu.sync_copy(rows, out_hbm.at[pl.ds(c * N + s * ROWS_PT + k * CH, CH)])
            return carry

        lax.fori_loop(0, INIT_CHUNKS, drain, 0)

    return prop


_deg_kernel = _build_deg()
_prop128 = _make_prop(128)
_prop64 = _make_prop(64)


_R = 1000
_GRID = N // _R


def _tc1_body(x_ref, w_ref, b_ref, d0_ref, d1_ref, y_ref, dis_ref):
    dis = lax.rsqrt(d0_ref[...] + d1_ref[...] + 1.0)
    h = jnp.dot(x_ref[...], w_ref[...], preferred_element_type=jnp.float32)
    h = (h + b_ref[...]) * dis
    y_ref[0] = h[:, :128]
    y_ref[1] = h[:, 128:]
    dis_ref[...] = dis


def _tc1(x, w1, b1, deg):
    return pl.pallas_call(
        _tc1_body,
        grid=(_GRID,),
        in_specs=[
            pl.BlockSpec((_R, IN_DIM), lambda i: (i, 0)),
            pl.BlockSpec((IN_DIM, HID_DIM), lambda i: (0, 0)),
            pl.BlockSpec((1, HID_DIM), lambda i: (0, 0)),
            pl.BlockSpec((_R, 1), lambda i: (i, 0)),
            pl.BlockSpec((_R, 1), lambda i: (i, 0)),
        ],
        out_specs=[
            pl.BlockSpec((NC, _R, 128), lambda i: (0, i, 0)),
            pl.BlockSpec((_R, 1), lambda i: (i, 0)),
        ],
        out_shape=[
            jax.ShapeDtypeStruct((NC, N, 128), jnp.float32),
            jax.ShapeDtypeStruct((N, 1), jnp.float32),
        ],
    )(x, w1, b1, deg[0].reshape(N, 1), deg[1].reshape(N, 1))


def _tc2_body(a_ref, w_ref, b_ref, dis_ref, y_ref):
    dis = dis_ref[...]
    h0 = jnp.maximum(a_ref[0] * dis, 0.0)
    h1 = jnp.maximum(a_ref[1] * dis, 0.0)
    y = jnp.dot(h0, w_ref[:128], preferred_element_type=jnp.float32)
    y = y + jnp.dot(h1, w_ref[128:], preferred_element_type=jnp.float32)
    y = (y + b_ref[...]) * dis
    y_ref[0] = y[:, :64]
    y_ref[1] = y[:, 64:]


def _tc2(acc1, w2, b2, dis):
    return pl.pallas_call(
        _tc2_body,
        grid=(_GRID,),
        in_specs=[
            pl.BlockSpec((NC, _R, 128), lambda i: (0, i, 0)),
            pl.BlockSpec((HID_DIM, LAT_DIM), lambda i: (0, 0)),
            pl.BlockSpec((1, LAT_DIM), lambda i: (0, 0)),
            pl.BlockSpec((_R, 1), lambda i: (i, 0)),
        ],
        out_specs=pl.BlockSpec((NC, _R, 64), lambda i: (0, i, 0)),
        out_shape=jax.ShapeDtypeStruct((NC, N, 64), jnp.float32),
    )(acc1, w2, b2, dis)


def _tc3_body(a_ref, w_ref, b_ref, dis_ref, y_ref):
    dis = dis_ref[...]
    z0 = a_ref[0] * dis
    z1 = a_ref[1] * dis
    y = jnp.dot(z0, w_ref[:64], preferred_element_type=jnp.float32)
    y = y + jnp.dot(z1, w_ref[64:], preferred_element_type=jnp.float32)
    y = (y + b_ref[...]) * dis
    y_ref[0] = y[:, :128]
    y_ref[1] = y[:, 128:]


def _tc3(acc2, w3, b3, dis):
    return pl.pallas_call(
        _tc3_body,
        grid=(_GRID,),
        in_specs=[
            pl.BlockSpec((NC, _R, 64), lambda i: (0, i, 0)),
            pl.BlockSpec((LAT_DIM, OUT_DIM), lambda i: (0, 0)),
            pl.BlockSpec((1, OUT_DIM), lambda i: (0, 0)),
            pl.BlockSpec((_R, 1), lambda i: (i, 0)),
        ],
        out_specs=pl.BlockSpec((NC, _R, 128), lambda i: (0, i, 0)),
        out_shape=jax.ShapeDtypeStruct((NC, N, 128), jnp.float32),
    )(acc2, w3, b3, dis)


def _tc4_body(a_ref, dis_ref, o_ref):
    dis = dis_ref[...]
    o_ref[:, :128] = a_ref[0] * dis
    o_ref[:, 128:] = a_ref[1] * dis


def _tc4(acc3, dis):
    return pl.pallas_call(
        _tc4_body,
        grid=(_GRID,),
        in_specs=[
            pl.BlockSpec((NC, _R, 128), lambda i: (0, i, 0)),
            pl.BlockSpec((_R, 1), lambda i: (i, 0)),
        ],
        out_specs=pl.BlockSpec((_R, OUT_DIM), lambda i: (i, 0)),
        out_shape=jax.ShapeDtypeStruct((N, OUT_DIM), jnp.float32),
    )(acc3, dis)


def kernel(x, edge_index, W1, b1, W2, b2, W3, b3):
    src = edge_index[0].astype(jnp.int32)
    dst = edge_index[1].astype(jnp.int32)
    dstb = dst.reshape(NT, NCHUNK, CH)
    srcb = jnp.stack([src, src + N]).reshape(NC, NT, NCHUNK, CH)
    dst_deg = dst.reshape(NC * NT, DEG_NCHUNK, DEG_CH)

    deg_parts = _deg_kernel(dst_deg).reshape(NC, N)

    y1, dis = _tc1(x, W1, b1.reshape(1, HID_DIM), deg_parts)
    acc1 = _prop128(y1.reshape(NC * N, 128), srcb, dstb)
    y2 = _tc2(acc1.reshape(NC, N, 128), W2, b2.reshape(1, LAT_DIM), dis)
    acc2 = _prop64(y2.reshape(NC * N, 64), srcb, dstb)
    y3 = _tc3(acc2.reshape(NC, N, 64), W3, b3.reshape(1, OUT_DIM), dis)
    acc3 = _prop128(y3.reshape(NC * N, 128), srcb, dstb)
    return _tc4(acc3.reshape(NC, N, 128), dis)
